# baseline (speedup 1.0000x reference)
import math

import jax
import jax.numpy as jnp
from jax import lax
from jax.experimental import pallas as pl
from jax.experimental.pallas import tpu as pltpu
from jax.experimental.pallas import tpu_sc as plsc

_L = 16
_LOG2T = 19
_T = 1 << _LOG2T
_GROWTH = math.exp((math.log(2048) - math.log(16)) / (_L - 1))
_RES = [math.floor(16 * math.pow(_GROWTH, i)) for i in range(_L)]
_MASK = _T - 1
_N = 524288
_P1 = -1640531535
_P2 = 805459861

_NC = 2
_NS = 16
_NW = _NC * _NS
_PPW = _N // _NW
_CHUNK = 256
_NCHUNK = _PPW // _CHUNK
_G = _CHUNK // 16
_NIDX = _CHUNK * 8
_GSUB = 8
_NSUB = _G // _GSUB
_SUBN = 8 * _GSUB * 16

def _body(x_hbm, y_hbm, z_hbm, tab0_hbm, tab1_hbm, scale_hbm, out_hbm,
          xv_, yv_, zv_, w0_, w1_, w2_, idxv, rows0, rows1, idxv2, rows0b,
          rows1b, xv2b0, xv2b1, xv2b2, outv, scal_v, spm5_0, spm5_1,
          sem0, sem1, sem0b, sem1b):
    cid = lax.axis_index("c")
    sid = lax.axis_index("s")
    wid = sid * _NC + cid
    pltpu.sync_copy(scale_hbm, scal_v)
    iota = lax.iota(jnp.int32, 16)
    iota32 = iota * 32

    _SL = _T // _NS
    soff = sid * _SL
    pltpu.sync_copy(tab0_hbm.at[pl.ds(5 * _T + soff, _SL)],
                    spm5_0.at[pl.ds(soff, _SL)])
    pltpu.sync_copy(tab1_hbm.at[pl.ds(5 * _T + soff, _SL)],
                    spm5_1.at[pl.ds(soff, _SL)])
    plsc.subcore_barrier()

    def chunk_body(ci, carry):
        cbase = (wid * _NCHUNK + ci) * _CHUNK
        pltpu.sync_copy(x_hbm.at[pl.ds(cbase, _CHUNK)], xv_)
        pltpu.sync_copy(y_hbm.at[pl.ds(cbase, _CHUNK)], yv_)
        pltpu.sync_copy(z_hbm.at[pl.ds(cbase, _CHUNK)], zv_)

        def phase_a(l, loff, bufs, srcs):
            w0x, w1x, w2x, idxx, r0x, r1x, s0x, s1x = bufs
            src0, src1 = srcs
            scale = plsc.load_gather(scal_v, [jnp.full((16,), l, jnp.int32)])
            cps = []
            for g in range(_G):
                xv = xv_[pl.ds(g * 16, 16)]
                yv = yv_[pl.ds(g * 16, 16)]
                zv = zv_[pl.ds(g * 16, 16)]
                tx = xv * scale
                ty = yv * scale
                tz = zv * scale
                ix = tx.astype(jnp.int32)
                iy = ty.astype(jnp.int32)
                iz = tz.astype(jnp.int32)
                w0x[pl.ds(g * 16, 16)] = tx - ix.astype(jnp.float32)
                w1x[pl.ds(g * 16, 16)] = ty - iy.astype(jnp.float32)
                w2x[pl.ds(g * 16, 16)] = tz - iz.astype(jnp.float32)
                a0 = ix
                a0b = ix + 1
                a1 = iy * _P1
                a1b = a1 + _P1
                a2 = iz * _P2
                a2b = a2 + _P2
                for c in range(8):
                    t0 = a0b if (c & 4) else a0
                    t1 = a1b if (c & 2) else a1
                    t2 = a2b if (c & 1) else a2
                    h = ((t0 ^ t1 ^ t2) & _MASK) + loff
                    sub, gg = g // _GSUB, g % _GSUB
                    j0 = sub * _SUBN + c * (_GSUB * 16) + gg * 16
                    idxx[pl.ds(j0, 16)] = h
                if (g + 1) % _GSUB == 0:
                    sub = g // _GSUB
                    lo = sub * _SUBN
                    cps.append((
                        pltpu.async_copy(src0.at[idxx.at[pl.ds(lo, _SUBN)]],
                                         r0x.at[pl.ds(lo, _SUBN)], s0x),
                        pltpu.async_copy(src1.at[idxx.at[pl.ds(lo, _SUBN)]],
                                         r1x.at[pl.ds(lo, _SUBN)], s1x)))
            return cps

        def phase_b(l, bufs, cps):
            w0x, w1x, w2x, idxx, r0x, r1x, s0x, s1x = bufs
            two_l = 2 * l
            for g in range(_G):
                sub, gg = g // _GSUB, g % _GSUB
                if gg == 0:
                    cps[sub][0].wait()
                    cps[sub][1].wait()
                dx = w0x[pl.ds(g * 16, 16)]
                dy = w1x[pl.ds(g * 16, 16)]
                dz = w2x[pl.ds(g * 16, 16)]
                obase = g * 512 + two_l
                for f, rows in ((0, r0x), (1, r1x)):
                    e = [rows[pl.ds(sub * _SUBN + c * (_GSUB * 16) + gg * 16, 16)]
                         for c in range(8)]
                    c00 = e[0] + dx * (e[4] - e[0])
                    c01 = e[1] + dx * (e[5] - e[1])
                    c10 = e[2] + dx * (e[6] - e[2])
                    c11 = e[3] + dx * (e[7] - e[3])
                    c0 = c00 + dy * (c10 - c00)
                    c1 = c01 + dy * (c11 - c01)
                    ov = c0 + dz * (c1 - c0)
                    plsc.store_scatter(outv, [iota32 + (obase + f)], ov)

        bufsA = (w0_, w1_, w2_, idxv, rows0, rows1, sem0, sem1)
        bufsB = (xv2b0, xv2b1, xv2b2, idxv2, rows0b, rows1b, sem0b, sem1b)

        def lvl_body(l, carry2):
            cps = phase_a(l, l * _T, bufsA, (tab0_hbm, tab1_hbm))
            phase_b(l, bufsA, cps)
            return carry2

        lax.fori_loop(0, 5, lvl_body, 0)
        cps5 = phase_a(5, 0, bufsB, (spm5_0, spm5_1))
        cps6 = phase_a(6, 6 * _T, bufsA, (tab0_hbm, tab1_hbm))
        phase_b(5, bufsB, cps5)
        phase_b(6, bufsA, cps6)
        lax.fori_loop(7, _L, lvl_body, 0)
        pltpu.sync_copy(outv, out_hbm.at[pl.ds(cbase * 32, _CHUNK * 32)])
        return carry

    lax.fori_loop(0, _NCHUNK, chunk_body, 0)


def kernel(xyz, tables):
    x, y, z = xyz[:, 0], xyz[:, 1], xyz[:, 2]
    tab0 = tables[:, :, 0].reshape(_L * _T)
    tab1 = tables[:, :, 1].reshape(_L * _T)
    scales = jnp.array([float(r) for r in _RES], jnp.float32)
    out = pl.kernel(
        _body,
        out_type=jax.ShapeDtypeStruct((_N * 32,), jnp.float32),
        mesh=plsc.VectorSubcoreMesh(core_axis_name="c", subcore_axis_name="s"),
        compiler_params=pltpu.CompilerParams(needs_layout_passes=False),
        scratch_types=[
            pltpu.VMEM((_CHUNK,), jnp.float32),
            pltpu.VMEM((_CHUNK,), jnp.float32),
            pltpu.VMEM((_CHUNK,), jnp.float32),
            pltpu.VMEM((_CHUNK,), jnp.float32),
            pltpu.VMEM((_CHUNK,), jnp.float32),
            pltpu.VMEM((_CHUNK,), jnp.float32),
            pltpu.VMEM((_NIDX,), jnp.int32),
            pltpu.VMEM((_NIDX,), jnp.float32),
            pltpu.VMEM((_NIDX,), jnp.float32),
            pltpu.VMEM((_NIDX,), jnp.int32),
            pltpu.VMEM((_NIDX,), jnp.float32),
            pltpu.VMEM((_NIDX,), jnp.float32),
            pltpu.VMEM((_CHUNK,), jnp.float32),
            pltpu.VMEM((_CHUNK,), jnp.float32),
            pltpu.VMEM((_CHUNK,), jnp.float32),
            pltpu.VMEM((_CHUNK * 32,), jnp.float32),
            pltpu.VMEM((_L,), jnp.float32),
            pltpu.VMEM_SHARED((_T,), jnp.float32),
            pltpu.VMEM_SHARED((_T,), jnp.float32),
            pltpu.SemaphoreType.DMA,
            pltpu.SemaphoreType.DMA,
            pltpu.SemaphoreType.DMA,
            pltpu.SemaphoreType.DMA,
        ],
    )(x, y, z, tab0, tab1, scales)
    return out.reshape(_N, 32)
